# baseline (speedup 1.0000x reference)
import numpy as np
import jax
import jax.numpy as jnp
from jax import lax
from jax.experimental import pallas as pl
from jax.experimental.pallas import tpu as pltpu, tpu_sc as plsc

_ANCHORS = np.array(
    [0.57273, 0.677385, 1.87446, 2.06253, 3.33843, 5.47434,
     7.88282, 3.52778, 9.77052, 9.16828], dtype=np.float32)
_AW = _ANCHORS[0::2]
_AH = _ANCHORS[1::2]
_NA = 5
_NC = 80
_NH = 19
_NW = 19
_NPIX = _NH * _NW
_NT = 50
_NTP = 64
_THRESH = 0.6
_OBJ = 5.0
_NFIELD = 16
_BIMG = 8


def _sig(v):
    return 1.0 / (1.0 + jnp.exp(-v))


def _iou(b1x, b1y, b1w, b1h, b2x, b2y, b2w, b2h):
    b1x1 = b1x - b1w / 2.0
    b1x2 = b1x + b1w / 2.0
    b1y1 = b1y - b1h / 2.0
    b1y2 = b1y + b1h / 2.0
    b2x1 = b2x - b2w / 2.0
    b2x2 = b2x + b2w / 2.0
    b2y1 = b2y - b2h / 2.0
    b2y2 = b2y + b2h / 2.0
    mx = jnp.minimum(b1x1, b2x1)
    Mx = jnp.maximum(b1x2, b2x2)
    my = jnp.minimum(b1y1, b2y1)
    My = jnp.maximum(b1y2, b2y2)
    cw = b1w + b2w - (Mx - mx)
    ch = b1h + b2h - (My - my)
    carea = jnp.where((cw <= 0) | (ch <= 0), 0.0, cw * ch)
    return carea / (b1w * b1h + b2w * b2h - carea)


def _sc_assign(tgt_hbm, out_row_hbm, out_col_hbm, tv, table, rrow, rcol):
    f32 = jnp.float32
    i32 = jnp.int32
    nb = out_row_hbm.shape[0]
    nflat = tgt_hbm.shape[1]
    sid = lax.axis_index("s")
    b = sid

    @pl.when(b < nb)
    def _():
        pltpu.sync_copy(tgt_hbm, tv)
        lane = lax.broadcasted_iota(i32, (16,), 0)
        row_b = jnp.full((16,), 0, i32) + b

        chunks = []
        carry_zeros = jnp.int32(0)
        for chunk in range(_NTP // 16):
            t = lane + (chunk * 16)
            base5 = jnp.minimum(t * 5, nflat - 5)
            cls_v = plsc.load_gather(tv, [row_b, base5 + 0])
            xs = plsc.load_gather(tv, [row_b, base5 + 1])
            ys = plsc.load_gather(tv, [row_b, base5 + 2])
            ws = plsc.load_gather(tv, [row_b, base5 + 3])
            hs = plsc.load_gather(tv, [row_b, base5 + 4])
            gx = xs * float(_NW)
            gy = ys * float(_NH)
            gw = ws * float(_NW)
            gh = hs * float(_NH)

            z = jnp.where(xs == 0.0, 1, 0).astype(i32)
            cz = plsc.cumsum(z)
            valid = ((cz + carry_zeros) == 0) & (t < _NT)
            carry_zeros = carry_zeros + jnp.sum(z)

            zero = jnp.zeros((16,), f32)
            best_v = jnp.full((16,), -jnp.inf, f32)
            best_n = jnp.zeros((16,), i32)
            aw_at = jnp.zeros((16,), f32)
            ah_at = jnp.zeros((16,), f32)
            for a in range(_NA):
                v = _iou(zero, zero, jnp.full((16,), _AW[a], f32),
                         jnp.full((16,), _AH[a], f32), zero, zero, gw, gh)
                take = v > best_v
                best_v = jnp.where(take, v, best_v)
                best_n = jnp.where(take, jnp.full((16,), a, i32), best_n)
                aw_at = jnp.where(take, jnp.full((16,), _AW[a], f32), aw_at)
                ah_at = jnp.where(take, jnp.full((16,), _AH[a], f32), ah_at)

            gi = gx.astype(i32)
            gj = gy.astype(i32)
            p = gj * _NW + gi
            cell = best_n * _NPIX + p

            for i in range(16):
                plsc.store_scatter(table, [cell], t,
                                   mask=(lane == i) & valid)
            chunks.append((t, valid, best_n, gi, gj, p, cell,
                           gx, gy, gw, gh, cls_v, aw_at, ah_at))

        for chunk, (t, valid, best_n, gi, gj, p, cell,
                    gx, gy, gw, gh, cls_v, aw_at, ah_at) in enumerate(chunks):
            last_t = plsc.load_gather(table, [cell])
            winner = valid & (last_t == t)
            gi_f = gi.astype(f32)
            gj_f = gj.astype(f32)
            row_fields = [
                (0, jnp.where(winner, 1.0, 0.0)),
                (2, gi_f),
                (3, gj_f),
                (6, gx),
                (7, gy),
                (8, gw),
                (9, gh),
                (10, cls_v),
                (11, aw_at),
                (12, ah_at),
                (13, gx - gi_f),
                (14, gy - gj_f),
            ]
            col_fields = [
                (1, best_n.astype(f32)),
                (4, p.astype(f32)),
                (5, jnp.where(valid, 1.0, 0.0)),
                (6, gx),
                (7, gy),
                (8, gw),
                (9, gh),
            ]
            for f, val in row_fields:
                rrow[f, pl.ds(chunk * 16, 16)] = val
            for f, val in col_fields:
                plsc.store_scatter(rcol, [t, jnp.full((16,), f, jnp.int32)],
                                   val)

        pltpu.sync_copy(rrow, out_row_hbm.at[b])
        pltpu.sync_copy(rcol, out_col_hbm.at[b])


def _run_sc_assign(tgt, nb):
    mesh = plsc.VectorSubcoreMesh(core_axis_name="c", subcore_axis_name="s",
                                  num_cores=1, num_subcores=16)
    return pl.kernel(
        _sc_assign,
        out_type=(
            jax.ShapeDtypeStruct((nb, _NFIELD, _NTP), jnp.float32),
            jax.ShapeDtypeStruct((nb, _NTP, _NFIELD), jnp.float32),
        ),
        mesh=mesh,
        compiler_params=pltpu.CompilerParams(needs_layout_passes=False),
        scratch_types=[
            pltpu.VMEM(tgt.shape, jnp.float32),
            pltpu.VMEM((_NA * _NPIX + 11,), jnp.int32),
            pltpu.VMEM((_NFIELD, _NTP), jnp.float32),
            pltpu.VMEM((_NTP, _NFIELD), jnp.float32),
        ],
    )(tgt)


def _image_loss(i, o_ref, grid_ref, rrow_ref, rcol_ref):
    f32 = jnp.float32
    i32 = jnp.int32

    winner_r = rrow_ref[i, 0:1, 0:_NT] > 0.5
    gi_r = rrow_ref[i, 2:3, 0:_NT]
    gj_r = rrow_ref[i, 3:4, 0:_NT]
    gx_r = rrow_ref[i, 6:7, 0:_NT]
    gy_r = rrow_ref[i, 7:8, 0:_NT]
    gw_r = rrow_ref[i, 8:9, 0:_NT]
    gh_r = rrow_ref[i, 9:10, 0:_NT]
    cls_r = rrow_ref[i, 10:11, 0:_NT]
    aw_r = rrow_ref[i, 11:12, 0:_NT]
    ah_r = rrow_ref[i, 12:13, 0:_NT]
    txw = rrow_ref[i, 13:14, 0:_NT]
    tyw = rrow_ref[i, 14:15, 0:_NT]

    n_c = rcol_ref[i, 0:_NT, 1:2]
    p_c = rcol_ref[i, 0:_NT, 4:5].astype(i32)
    valid_c = rcol_ref[i, 0:_NT, 5:6] > 0.5
    gx_c = rcol_ref[i, 0:_NT, 6:7]
    gy_c = rcol_ref[i, 0:_NT, 7:8]
    gw_c = rcol_ref[i, 0:_NT, 8:9]
    gh_c = rcol_ref[i, 0:_NT, 9:10]

    gxgrid = grid_ref[0:1, :]
    gygrid = grid_ref[1:2, :]
    p_io = lax.broadcasted_iota(i32, (_NT, _NPIX), 1)

    acc_xy = jnp.zeros((1, _NPIX), f32)
    acc_wh = jnp.zeros((1, _NPIX), f32)
    acc_conf = jnp.zeros((1, _NPIX), f32)
    G = jnp.zeros((8, _NT), f32)
    Lg = jnp.zeros((_NC, _NT), f32)

    b2x1 = gx_c - gw_c / 2.0
    b2x2 = gx_c + gw_c / 2.0
    b2y1 = gy_c - gh_c / 2.0
    b2y2 = gy_c + gh_c / 2.0
    b2area = gw_c * gh_c
    rhs_row = jnp.where(valid_c, _THRESH * b2area, jnp.inf)
    tiles = [(0, 16), (16, 32), (32, 48), (48, _NT)]

    msks = [jnp.where((p_io == p_c) & (n_c == float(a)), 1.0, 0.0)
            for a in range(_NA)]

    for a in range(_NA):
        base = a * (5 + _NC)
        x_a = o_ref[i, base + 0:base + 1, :]
        y_a = o_ref[i, base + 1:base + 2, :]
        w_a = o_ref[i, base + 2:base + 3, :]
        h_a = o_ref[i, base + 3:base + 4, :]
        c_a = o_ref[i, base + 4:base + 5, :]
        sigx = _sig(x_a); sigy = _sig(y_a); sigc = _sig(c_a)
        pxc = sigx + gxgrid
        pyc = sigy + gygrid
        pw = jnp.exp(w_a) * _AW[a]
        ph = jnp.exp(h_a) * _AH[a]

        acc_xy += (sigx - 0.5) ** 2 + (sigy - 0.5) ** 2
        acc_wh += w_a * w_a + h_a * h_a

        b1x1 = pxc - pw / 2.0
        b1x2 = pxc + pw / 2.0
        b1y1 = pyc - ph / 2.0
        b1y2 = pyc + ph / 2.0
        lhs_off = _THRESH * (pw * ph)
        diffmax = jnp.full((1, _NPIX), -jnp.inf, f32)
        for lo, hi in tiles:
            cw = (jnp.minimum(b1x2, b2x2[lo:hi])
                  - jnp.maximum(b1x1, b2x1[lo:hi]))
            ch = (jnp.minimum(b1y2, b2y2[lo:hi])
                  - jnp.maximum(b1y1, b2y1[lo:hi]))
            carea = jnp.maximum(cw, 0.0) * jnp.maximum(ch, 0.0)
            diff = (carea * (1.0 + _THRESH) - lhs_off) - rhs_row[lo:hi]
            diffmax = jnp.maximum(diffmax,
                                  jnp.max(diff, axis=0, keepdims=True))
        mask0_a = jnp.where(diffmax > 0.0, 0.0, 1.0)
        acc_conf += mask0_a * sigc * sigc

        mskf = msks[a]
        F_a = jnp.concatenate([sigx, sigy, w_a, h_a, sigc, pw, ph, mask0_a],
                              axis=0)
        G += lax.dot_general(F_a, mskf, (((1,), (1,)), ((), ())),
                             preferred_element_type=f32)

        cls_a = o_ref[i, base + 5:base + 5 + _NC, :]
        Lg += lax.dot_general(cls_a, mskf, (((1,), (1,)), ((), ())),
                              preferred_element_type=f32)

    g_sigx = G[0:1, :]
    g_sigy = G[1:2, :]
    g_w = G[2:3, :]
    g_h = G[3:4, :]
    g_conf = G[4:5, :]
    g_pw = G[5:6, :]
    g_ph = G[6:7, :]
    mask0_at = G[7:8, :]

    tww = jnp.log(gw_r / aw_r)
    thw = jnp.log(gh_r / ah_r)
    pxc_at = g_sigx + gi_r
    pyc_at = g_sigy + gj_r
    iou_at = _iou(gx_r, gy_r, gw_r, gh_r, pxc_at, pyc_at, g_pw, g_ph)

    corr = ((g_sigx - txw) ** 2 - (g_sigx - 0.5) ** 2
            + (g_sigy - tyw) ** 2 - (g_sigy - 0.5) ** 2
            + (g_w - tww) ** 2 - g_w * g_w
            + (g_h - thw) ** 2 - g_h * g_h
            + _OBJ * (g_conf - iou_at) ** 2 - mask0_at * g_conf * g_conf)
    corr_sum = jnp.sum(jnp.where(winner_r, corr, 0.0))

    cint = cls_r.astype(i32)
    c_io = lax.broadcasted_iota(i32, (_NC, _NT), 0)
    pick = jnp.sum(jnp.where(c_io == cint, Lg, 0.0), axis=0, keepdims=True)
    m = jnp.max(Lg, axis=0, keepdims=True)
    lse = m + jnp.log(jnp.sum(jnp.exp(Lg - m), axis=0, keepdims=True))
    nll = lse - pick
    cls_sum = jnp.sum(jnp.where(winner_r, nll, 0.0))

    dense_sum = jnp.sum(acc_xy) + jnp.sum(acc_wh) + jnp.sum(acc_conf)
    return (dense_sum + corr_sum) * 0.5 + cls_sum


def _yolo_kernel(o_ref, grid_ref, rrow_ref, rcol_ref, out_ref):
    total = jnp.float32(0.0)
    for i in range(_BIMG):
        total = total + _image_loss(i, o_ref, grid_ref, rrow_ref, rcol_ref)

    @pl.when(pl.program_id(0) == 0)
    def _init():
        out_ref[0] = jnp.zeros((1, 1), jnp.float32)

    out_ref[0] = out_ref[0] + jnp.full((1, 1), total, jnp.float32)


def _grid_consts():
    p = np.arange(_NPIX)
    return np.stack([(p % _NW).astype(np.float32),
                     (p // _NW).astype(np.float32)], axis=0)


def kernel(output, target):
    nB = output.shape[0]
    o = output.reshape(nB, _NA * (5 + _NC), _NPIX)
    gridc = jnp.asarray(_grid_consts())

    rec_row, rec_col = _run_sc_assign(target, nB)

    partial = pl.pallas_call(
        _yolo_kernel,
        grid=(nB // _BIMG,),
        in_specs=[
            pl.BlockSpec((_BIMG, _NA * (5 + _NC), _NPIX),
                         lambda b: (b, 0, 0)),
            pl.BlockSpec((2, _NPIX), lambda b: (0, 0)),
            pl.BlockSpec((_BIMG, _NFIELD, _NTP), lambda b: (b, 0, 0)),
            pl.BlockSpec((_BIMG, _NTP, _NFIELD), lambda b: (b, 0, 0)),
        ],
        out_specs=pl.BlockSpec((1, 1, 1), lambda b: (0, 0, 0)),
        out_shape=jax.ShapeDtypeStruct((1, 1, 1), jnp.float32),
    )(o, gridc, rec_row, rec_col)
    return partial.reshape(())


# speedup vs baseline: 1.0053x; 1.0053x over previous
import numpy as np
import jax
import jax.numpy as jnp
from jax import lax
from jax.experimental import pallas as pl
from jax.experimental.pallas import tpu as pltpu, tpu_sc as plsc

_ANCHORS = np.array(
    [0.57273, 0.677385, 1.87446, 2.06253, 3.33843, 5.47434,
     7.88282, 3.52778, 9.77052, 9.16828], dtype=np.float32)
_AW = _ANCHORS[0::2]
_AH = _ANCHORS[1::2]
_NA = 5
_NC = 80
_NH = 19
_NW = 19
_NPIX = _NH * _NW
_NT = 50
_NTP = 64
_THRESH = 0.6
_OBJ = 5.0
_NFIELD = 16
_BIMG = 2


def _sig(v):
    return 1.0 / (1.0 + jnp.exp(-v))


def _iou(b1x, b1y, b1w, b1h, b2x, b2y, b2w, b2h):
    b1x1 = b1x - b1w / 2.0
    b1x2 = b1x + b1w / 2.0
    b1y1 = b1y - b1h / 2.0
    b1y2 = b1y + b1h / 2.0
    b2x1 = b2x - b2w / 2.0
    b2x2 = b2x + b2w / 2.0
    b2y1 = b2y - b2h / 2.0
    b2y2 = b2y + b2h / 2.0
    mx = jnp.minimum(b1x1, b2x1)
    Mx = jnp.maximum(b1x2, b2x2)
    my = jnp.minimum(b1y1, b2y1)
    My = jnp.maximum(b1y2, b2y2)
    cw = b1w + b2w - (Mx - mx)
    ch = b1h + b2h - (My - my)
    carea = jnp.where((cw <= 0) | (ch <= 0), 0.0, cw * ch)
    return carea / (b1w * b1h + b2w * b2h - carea)


def _sc_assign(tgt_hbm, out_row_hbm, out_col_hbm, tv, table, rrow, rcol):
    f32 = jnp.float32
    i32 = jnp.int32
    nb = out_row_hbm.shape[0]
    nflat = tgt_hbm.shape[1]
    sid = lax.axis_index("s")
    b = sid

    @pl.when(b < nb)
    def _():
        pltpu.sync_copy(tgt_hbm, tv)
        lane = lax.broadcasted_iota(i32, (16,), 0)
        row_b = jnp.full((16,), 0, i32) + b

        chunks = []
        carry_zeros = jnp.int32(0)
        for chunk in range(_NTP // 16):
            t = lane + (chunk * 16)
            base5 = jnp.minimum(t * 5, nflat - 5)
            cls_v = plsc.load_gather(tv, [row_b, base5 + 0])
            xs = plsc.load_gather(tv, [row_b, base5 + 1])
            ys = plsc.load_gather(tv, [row_b, base5 + 2])
            ws = plsc.load_gather(tv, [row_b, base5 + 3])
            hs = plsc.load_gather(tv, [row_b, base5 + 4])
            gx = xs * float(_NW)
            gy = ys * float(_NH)
            gw = ws * float(_NW)
            gh = hs * float(_NH)

            z = jnp.where(xs == 0.0, 1, 0).astype(i32)
            cz = plsc.cumsum(z)
            valid = ((cz + carry_zeros) == 0) & (t < _NT)
            carry_zeros = carry_zeros + jnp.sum(z)

            zero = jnp.zeros((16,), f32)
            best_v = jnp.full((16,), -jnp.inf, f32)
            best_n = jnp.zeros((16,), i32)
            aw_at = jnp.zeros((16,), f32)
            ah_at = jnp.zeros((16,), f32)
            for a in range(_NA):
                v = _iou(zero, zero, jnp.full((16,), _AW[a], f32),
                         jnp.full((16,), _AH[a], f32), zero, zero, gw, gh)
                take = v > best_v
                best_v = jnp.where(take, v, best_v)
                best_n = jnp.where(take, jnp.full((16,), a, i32), best_n)
                aw_at = jnp.where(take, jnp.full((16,), _AW[a], f32), aw_at)
                ah_at = jnp.where(take, jnp.full((16,), _AH[a], f32), ah_at)

            gi = gx.astype(i32)
            gj = gy.astype(i32)
            p = gj * _NW + gi
            cell = best_n * _NPIX + p

            for i in range(16):
                plsc.store_scatter(table, [cell], t,
                                   mask=(lane == i) & valid)
            chunks.append((t, valid, best_n, gi, gj, p, cell,
                           gx, gy, gw, gh, cls_v, aw_at, ah_at))

        for chunk, (t, valid, best_n, gi, gj, p, cell,
                    gx, gy, gw, gh, cls_v, aw_at, ah_at) in enumerate(chunks):
            last_t = plsc.load_gather(table, [cell])
            winner = valid & (last_t == t)
            gi_f = gi.astype(f32)
            gj_f = gj.astype(f32)
            row_fields = [
                (0, jnp.where(winner, 1.0, 0.0)),
                (2, gi_f),
                (3, gj_f),
                (6, gx),
                (7, gy),
                (8, gw),
                (9, gh),
                (10, cls_v),
                (11, aw_at),
                (12, ah_at),
                (13, gx - gi_f),
                (14, gy - gj_f),
            ]
            col_fields = [
                (1, best_n.astype(f32)),
                (4, p.astype(f32)),
                (5, jnp.where(valid, 1.0, 0.0)),
                (6, gx),
                (7, gy),
                (8, gw),
                (9, gh),
            ]
            for f, val in row_fields:
                rrow[f, pl.ds(chunk * 16, 16)] = val
            for f, val in col_fields:
                plsc.store_scatter(rcol, [t, jnp.full((16,), f, jnp.int32)],
                                   val)

        pltpu.sync_copy(rrow, out_row_hbm.at[b])
        pltpu.sync_copy(rcol, out_col_hbm.at[b])


def _run_sc_assign(tgt, nb):
    mesh = plsc.VectorSubcoreMesh(core_axis_name="c", subcore_axis_name="s",
                                  num_cores=1, num_subcores=16)
    return pl.kernel(
        _sc_assign,
        out_type=(
            jax.ShapeDtypeStruct((nb, _NFIELD, _NTP), jnp.float32),
            jax.ShapeDtypeStruct((nb, _NTP, _NFIELD), jnp.float32),
        ),
        mesh=mesh,
        compiler_params=pltpu.CompilerParams(needs_layout_passes=False),
        scratch_types=[
            pltpu.VMEM(tgt.shape, jnp.float32),
            pltpu.VMEM((_NA * _NPIX + 11,), jnp.int32),
            pltpu.VMEM((_NFIELD, _NTP), jnp.float32),
            pltpu.VMEM((_NTP, _NFIELD), jnp.float32),
        ],
    )(tgt)


def _image_loss(i, o_ref, grid_ref, rrow_ref, rcol_ref):
    f32 = jnp.float32
    i32 = jnp.int32

    winner_r = rrow_ref[i, 0:1, 0:_NT] > 0.5
    gi_r = rrow_ref[i, 2:3, 0:_NT]
    gj_r = rrow_ref[i, 3:4, 0:_NT]
    gx_r = rrow_ref[i, 6:7, 0:_NT]
    gy_r = rrow_ref[i, 7:8, 0:_NT]
    gw_r = rrow_ref[i, 8:9, 0:_NT]
    gh_r = rrow_ref[i, 9:10, 0:_NT]
    cls_r = rrow_ref[i, 10:11, 0:_NT]
    aw_r = rrow_ref[i, 11:12, 0:_NT]
    ah_r = rrow_ref[i, 12:13, 0:_NT]
    txw = rrow_ref[i, 13:14, 0:_NT]
    tyw = rrow_ref[i, 14:15, 0:_NT]

    n_c = rcol_ref[i, 0:_NT, 1:2]
    p_c = rcol_ref[i, 0:_NT, 4:5].astype(i32)
    valid_c = rcol_ref[i, 0:_NT, 5:6] > 0.5
    gx_c = rcol_ref[i, 0:_NT, 6:7]
    gy_c = rcol_ref[i, 0:_NT, 7:8]
    gw_c = rcol_ref[i, 0:_NT, 8:9]
    gh_c = rcol_ref[i, 0:_NT, 9:10]

    gxgrid = grid_ref[0:1, :]
    gygrid = grid_ref[1:2, :]
    p_io = lax.broadcasted_iota(i32, (_NT, _NPIX), 1)

    acc_xy = jnp.zeros((1, _NPIX), f32)
    acc_wh = jnp.zeros((1, _NPIX), f32)
    acc_conf = jnp.zeros((1, _NPIX), f32)
    G = jnp.zeros((8, _NT), f32)
    Lg = jnp.zeros((_NC, _NT), f32)

    b2x1 = gx_c - gw_c / 2.0
    b2x2 = gx_c + gw_c / 2.0
    b2y1 = gy_c - gh_c / 2.0
    b2y2 = gy_c + gh_c / 2.0
    b2area = gw_c * gh_c
    rhs_row = jnp.where(valid_c, _THRESH * b2area, jnp.inf)
    tiles = [(0, 16), (16, 32), (32, 48), (48, _NT)]

    msks = [jnp.where((p_io == p_c) & (n_c == float(a)), 1.0, 0.0)
            for a in range(_NA)]

    for a in range(_NA):
        base = a * (5 + _NC)
        x_a = o_ref[i, base + 0:base + 1, :]
        y_a = o_ref[i, base + 1:base + 2, :]
        w_a = o_ref[i, base + 2:base + 3, :]
        h_a = o_ref[i, base + 3:base + 4, :]
        c_a = o_ref[i, base + 4:base + 5, :]
        sigx = _sig(x_a); sigy = _sig(y_a); sigc = _sig(c_a)
        pxc = sigx + gxgrid
        pyc = sigy + gygrid
        pw = jnp.exp(w_a) * _AW[a]
        ph = jnp.exp(h_a) * _AH[a]

        acc_xy += (sigx - 0.5) ** 2 + (sigy - 0.5) ** 2
        acc_wh += w_a * w_a + h_a * h_a

        b1x1 = pxc - pw / 2.0
        b1x2 = pxc + pw / 2.0
        b1y1 = pyc - ph / 2.0
        b1y2 = pyc + ph / 2.0
        lhs_off = _THRESH * (pw * ph)
        diffmax = jnp.full((1, _NPIX), -jnp.inf, f32)
        for lo, hi in tiles:
            cw = (jnp.minimum(b1x2, b2x2[lo:hi])
                  - jnp.maximum(b1x1, b2x1[lo:hi]))
            ch = (jnp.minimum(b1y2, b2y2[lo:hi])
                  - jnp.maximum(b1y1, b2y1[lo:hi]))
            carea = jnp.maximum(cw, 0.0) * jnp.maximum(ch, 0.0)
            diff = (carea * (1.0 + _THRESH) - lhs_off) - rhs_row[lo:hi]
            diffmax = jnp.maximum(diffmax,
                                  jnp.max(diff, axis=0, keepdims=True))
        mask0_a = jnp.where(diffmax > 0.0, 0.0, 1.0)
        acc_conf += mask0_a * sigc * sigc

        mskf = msks[a]
        F_a = jnp.concatenate([sigx, sigy, w_a, h_a, sigc, pw, ph, mask0_a],
                              axis=0)
        G += lax.dot_general(F_a, mskf, (((1,), (1,)), ((), ())),
                             preferred_element_type=f32)

        cls_a = o_ref[i, base + 5:base + 5 + _NC, :]
        Lg += lax.dot_general(cls_a, mskf, (((1,), (1,)), ((), ())),
                              preferred_element_type=f32)

    g_sigx = G[0:1, :]
    g_sigy = G[1:2, :]
    g_w = G[2:3, :]
    g_h = G[3:4, :]
    g_conf = G[4:5, :]
    g_pw = G[5:6, :]
    g_ph = G[6:7, :]
    mask0_at = G[7:8, :]

    tww = jnp.log(gw_r / aw_r)
    thw = jnp.log(gh_r / ah_r)
    pxc_at = g_sigx + gi_r
    pyc_at = g_sigy + gj_r
    iou_at = _iou(gx_r, gy_r, gw_r, gh_r, pxc_at, pyc_at, g_pw, g_ph)

    corr = ((g_sigx - txw) ** 2 - (g_sigx - 0.5) ** 2
            + (g_sigy - tyw) ** 2 - (g_sigy - 0.5) ** 2
            + (g_w - tww) ** 2 - g_w * g_w
            + (g_h - thw) ** 2 - g_h * g_h
            + _OBJ * (g_conf - iou_at) ** 2 - mask0_at * g_conf * g_conf)
    corr_sum = jnp.sum(jnp.where(winner_r, corr, 0.0))

    cint = cls_r.astype(i32)
    c_io = lax.broadcasted_iota(i32, (_NC, _NT), 0)
    pick = jnp.sum(jnp.where(c_io == cint, Lg, 0.0), axis=0, keepdims=True)
    m = jnp.max(Lg, axis=0, keepdims=True)
    lse = m + jnp.log(jnp.sum(jnp.exp(Lg - m), axis=0, keepdims=True))
    nll = lse - pick
    cls_sum = jnp.sum(jnp.where(winner_r, nll, 0.0))

    dense_sum = jnp.sum(acc_xy) + jnp.sum(acc_wh) + jnp.sum(acc_conf)
    return (dense_sum + corr_sum) * 0.5 + cls_sum


def _yolo_kernel(o_ref, grid_ref, rrow_ref, rcol_ref, out_ref):
    total = jnp.float32(0.0)
    for i in range(_BIMG):
        total = total + _image_loss(i, o_ref, grid_ref, rrow_ref, rcol_ref)

    @pl.when(pl.program_id(0) == 0)
    def _init():
        out_ref[0] = jnp.zeros((1, 1), jnp.float32)

    out_ref[0] = out_ref[0] + jnp.full((1, 1), total, jnp.float32)


def _grid_consts():
    p = np.arange(_NPIX)
    return np.stack([(p % _NW).astype(np.float32),
                     (p // _NW).astype(np.float32)], axis=0)


def kernel(output, target):
    nB = output.shape[0]
    o = output.reshape(nB, _NA * (5 + _NC), _NPIX)
    gridc = jnp.asarray(_grid_consts())

    rec_row, rec_col = _run_sc_assign(target, nB)

    partial = pl.pallas_call(
        _yolo_kernel,
        grid=(nB // _BIMG,),
        in_specs=[
            pl.BlockSpec((_BIMG, _NA * (5 + _NC), _NPIX),
                         lambda b: (b, 0, 0)),
            pl.BlockSpec((2, _NPIX), lambda b: (0, 0)),
            pl.BlockSpec((_BIMG, _NFIELD, _NTP), lambda b: (b, 0, 0)),
            pl.BlockSpec((_BIMG, _NTP, _NFIELD), lambda b: (b, 0, 0)),
        ],
        out_specs=pl.BlockSpec((1, 1, 1), lambda b: (0, 0, 0)),
        out_shape=jax.ShapeDtypeStruct((1, 1, 1), jnp.float32),
    )(o, gridc, rec_row, rec_col)
    return partial.reshape(())


# speedup vs baseline: 1.0243x; 1.0189x over previous
import numpy as np
import jax
import jax.numpy as jnp
from jax import lax
from jax.experimental import pallas as pl
from jax.experimental.pallas import tpu as pltpu, tpu_sc as plsc

_ANCHORS = np.array(
    [0.57273, 0.677385, 1.87446, 2.06253, 3.33843, 5.47434,
     7.88282, 3.52778, 9.77052, 9.16828], dtype=np.float32)
_AW = _ANCHORS[0::2]
_AH = _ANCHORS[1::2]
_NA = 5
_NC = 80
_NH = 19
_NW = 19
_NPIX = _NH * _NW
_NT = 50
_NTP = 64
_THRESH = 0.6
_OBJ = 5.0
_NFIELD = 16
_BIMG = 4


def _sig(v):
    return 1.0 / (1.0 + jnp.exp(-v))


def _iou(b1x, b1y, b1w, b1h, b2x, b2y, b2w, b2h):
    b1x1 = b1x - b1w / 2.0
    b1x2 = b1x + b1w / 2.0
    b1y1 = b1y - b1h / 2.0
    b1y2 = b1y + b1h / 2.0
    b2x1 = b2x - b2w / 2.0
    b2x2 = b2x + b2w / 2.0
    b2y1 = b2y - b2h / 2.0
    b2y2 = b2y + b2h / 2.0
    mx = jnp.minimum(b1x1, b2x1)
    Mx = jnp.maximum(b1x2, b2x2)
    my = jnp.minimum(b1y1, b2y1)
    My = jnp.maximum(b1y2, b2y2)
    cw = b1w + b2w - (Mx - mx)
    ch = b1h + b2h - (My - my)
    carea = jnp.where((cw <= 0) | (ch <= 0), 0.0, cw * ch)
    return carea / (b1w * b1h + b2w * b2h - carea)


def _sc_assign(tgt_hbm, out_row_hbm, out_col_hbm, tv, table, rrow, rcol):
    f32 = jnp.float32
    i32 = jnp.int32
    nb = out_row_hbm.shape[0]
    nflat = tgt_hbm.shape[1]
    sid = lax.axis_index("s")
    b = sid

    @pl.when(b < nb)
    def _():
        pltpu.sync_copy(tgt_hbm, tv)
        lane = lax.broadcasted_iota(i32, (16,), 0)
        row_b = jnp.full((16,), 0, i32) + b

        chunks = []
        carry_zeros = jnp.int32(0)
        for chunk in range(_NTP // 16):
            t = lane + (chunk * 16)
            base5 = jnp.minimum(t * 5, nflat - 5)
            cls_v = plsc.load_gather(tv, [row_b, base5 + 0])
            xs = plsc.load_gather(tv, [row_b, base5 + 1])
            ys = plsc.load_gather(tv, [row_b, base5 + 2])
            ws = plsc.load_gather(tv, [row_b, base5 + 3])
            hs = plsc.load_gather(tv, [row_b, base5 + 4])
            gx = xs * float(_NW)
            gy = ys * float(_NH)
            gw = ws * float(_NW)
            gh = hs * float(_NH)

            z = jnp.where(xs == 0.0, 1, 0).astype(i32)
            cz = plsc.cumsum(z)
            valid = ((cz + carry_zeros) == 0) & (t < _NT)
            carry_zeros = carry_zeros + jnp.sum(z)

            zero = jnp.zeros((16,), f32)
            best_v = jnp.full((16,), -jnp.inf, f32)
            best_n = jnp.zeros((16,), i32)
            aw_at = jnp.zeros((16,), f32)
            ah_at = jnp.zeros((16,), f32)
            for a in range(_NA):
                v = _iou(zero, zero, jnp.full((16,), _AW[a], f32),
                         jnp.full((16,), _AH[a], f32), zero, zero, gw, gh)
                take = v > best_v
                best_v = jnp.where(take, v, best_v)
                best_n = jnp.where(take, jnp.full((16,), a, i32), best_n)
                aw_at = jnp.where(take, jnp.full((16,), _AW[a], f32), aw_at)
                ah_at = jnp.where(take, jnp.full((16,), _AH[a], f32), ah_at)

            gi = gx.astype(i32)
            gj = gy.astype(i32)
            p = gj * _NW + gi
            cell = best_n * _NPIX + p

            for i in range(16):
                plsc.store_scatter(table, [cell], t,
                                   mask=(lane == i) & valid)
            chunks.append((t, valid, best_n, gi, gj, p, cell,
                           gx, gy, gw, gh, cls_v, aw_at, ah_at))

        for chunk, (t, valid, best_n, gi, gj, p, cell,
                    gx, gy, gw, gh, cls_v, aw_at, ah_at) in enumerate(chunks):
            last_t = plsc.load_gather(table, [cell])
            winner = valid & (last_t == t)
            gi_f = gi.astype(f32)
            gj_f = gj.astype(f32)
            row_fields = [
                (0, jnp.where(winner, 1.0, 0.0)),
                (2, gi_f),
                (3, gj_f),
                (6, gx),
                (7, gy),
                (8, gw),
                (9, gh),
                (10, cls_v),
                (11, aw_at),
                (12, ah_at),
                (13, gx - gi_f),
                (14, gy - gj_f),
            ]
            col_fields = [
                (1, best_n.astype(f32)),
                (4, p.astype(f32)),
                (5, jnp.where(valid, 1.0, 0.0)),
                (6, gx),
                (7, gy),
                (8, gw),
                (9, gh),
            ]
            for f, val in row_fields:
                rrow[f, pl.ds(chunk * 16, 16)] = val
            for f, val in col_fields:
                plsc.store_scatter(rcol, [t, jnp.full((16,), f, jnp.int32)],
                                   val)

        pltpu.sync_copy(rrow, out_row_hbm.at[b])
        pltpu.sync_copy(rcol, out_col_hbm.at[b])


def _run_sc_assign(tgt, nb):
    mesh = plsc.VectorSubcoreMesh(core_axis_name="c", subcore_axis_name="s",
                                  num_cores=1, num_subcores=16)
    return pl.kernel(
        _sc_assign,
        out_type=(
            jax.ShapeDtypeStruct((nb, _NFIELD, _NTP), jnp.float32),
            jax.ShapeDtypeStruct((nb, _NTP, _NFIELD), jnp.float32),
        ),
        mesh=mesh,
        compiler_params=pltpu.CompilerParams(needs_layout_passes=False),
        scratch_types=[
            pltpu.VMEM(tgt.shape, jnp.float32),
            pltpu.VMEM((_NA * _NPIX + 11,), jnp.int32),
            pltpu.VMEM((_NFIELD, _NTP), jnp.float32),
            pltpu.VMEM((_NTP, _NFIELD), jnp.float32),
        ],
    )(tgt)


def _image_loss(i, o_ref, grid_ref, rrow_ref, rcol_ref):
    f32 = jnp.float32
    i32 = jnp.int32

    winner_r = rrow_ref[i, 0:1, 0:_NT] > 0.5
    gi_r = rrow_ref[i, 2:3, 0:_NT]
    gj_r = rrow_ref[i, 3:4, 0:_NT]
    gx_r = rrow_ref[i, 6:7, 0:_NT]
    gy_r = rrow_ref[i, 7:8, 0:_NT]
    gw_r = rrow_ref[i, 8:9, 0:_NT]
    gh_r = rrow_ref[i, 9:10, 0:_NT]
    cls_r = rrow_ref[i, 10:11, 0:_NT]
    aw_r = rrow_ref[i, 11:12, 0:_NT]
    ah_r = rrow_ref[i, 12:13, 0:_NT]
    txw = rrow_ref[i, 13:14, 0:_NT]
    tyw = rrow_ref[i, 14:15, 0:_NT]

    n_c = rcol_ref[i, 0:_NT, 1:2]
    p_c = rcol_ref[i, 0:_NT, 4:5].astype(i32)
    valid_c = rcol_ref[i, 0:_NT, 5:6] > 0.5
    gx_c = rcol_ref[i, 0:_NT, 6:7]
    gy_c = rcol_ref[i, 0:_NT, 7:8]
    gw_c = rcol_ref[i, 0:_NT, 8:9]
    gh_c = rcol_ref[i, 0:_NT, 9:10]

    gxgrid = grid_ref[0:1, :]
    gygrid = grid_ref[1:2, :]
    p_io = lax.broadcasted_iota(i32, (_NT, _NPIX), 1)

    acc_xy = jnp.zeros((1, _NPIX), f32)
    acc_wh = jnp.zeros((1, _NPIX), f32)
    acc_conf = jnp.zeros((1, _NPIX), f32)
    G = jnp.zeros((8, _NT), f32)
    Lg = jnp.zeros((_NC, _NT), f32)

    b2x1 = gx_c - gw_c / 2.0
    b2x2 = gx_c + gw_c / 2.0
    b2y1 = gy_c - gh_c / 2.0
    b2y2 = gy_c + gh_c / 2.0
    b2area = gw_c * gh_c
    rhs_row = jnp.where(valid_c, _THRESH * b2area, jnp.inf)
    tiles = [(0, 16), (16, 32), (32, 48), (48, _NT)]

    msks = [jnp.where((p_io == p_c) & (n_c == float(a)), 1.0, 0.0)
            for a in range(_NA)]

    for a in range(_NA):
        base = a * (5 + _NC)
        x_a = o_ref[i, base + 0:base + 1, :]
        y_a = o_ref[i, base + 1:base + 2, :]
        w_a = o_ref[i, base + 2:base + 3, :]
        h_a = o_ref[i, base + 3:base + 4, :]
        c_a = o_ref[i, base + 4:base + 5, :]
        sigx = _sig(x_a); sigy = _sig(y_a); sigc = _sig(c_a)
        pxc = sigx + gxgrid
        pyc = sigy + gygrid
        pw = jnp.exp(w_a) * _AW[a]
        ph = jnp.exp(h_a) * _AH[a]

        acc_xy += (sigx - 0.5) ** 2 + (sigy - 0.5) ** 2
        acc_wh += w_a * w_a + h_a * h_a

        b1x1 = pxc - pw / 2.0
        b1x2 = pxc + pw / 2.0
        b1y1 = pyc - ph / 2.0
        b1y2 = pyc + ph / 2.0
        lhs_off = _THRESH * (pw * ph)
        diffmax = jnp.full((1, _NPIX), -jnp.inf, f32)
        for lo, hi in tiles:
            cw = (jnp.minimum(b1x2, b2x2[lo:hi])
                  - jnp.maximum(b1x1, b2x1[lo:hi]))
            ch = (jnp.minimum(b1y2, b2y2[lo:hi])
                  - jnp.maximum(b1y1, b2y1[lo:hi]))
            carea = jnp.maximum(cw, 0.0) * jnp.maximum(ch, 0.0)
            diff = (carea * (1.0 + _THRESH) - lhs_off) - rhs_row[lo:hi]
            diffmax = jnp.maximum(diffmax,
                                  jnp.max(diff, axis=0, keepdims=True))
        mask0_a = jnp.where(diffmax > 0.0, 0.0, 1.0)
        acc_conf += mask0_a * sigc * sigc

        mskf = msks[a]
        F_a = jnp.concatenate([sigx, sigy, w_a, h_a, sigc, pw, ph, mask0_a],
                              axis=0)
        G += lax.dot_general(F_a, mskf, (((1,), (1,)), ((), ())),
                             preferred_element_type=f32)

        cls_a = o_ref[i, base + 5:base + 5 + _NC, :]
        Lg += lax.dot_general(cls_a, mskf, (((1,), (1,)), ((), ())),
                              preferred_element_type=f32)

    g_sigx = G[0:1, :]
    g_sigy = G[1:2, :]
    g_w = G[2:3, :]
    g_h = G[3:4, :]
    g_conf = G[4:5, :]
    g_pw = G[5:6, :]
    g_ph = G[6:7, :]
    mask0_at = G[7:8, :]

    tww = jnp.log(gw_r / aw_r)
    thw = jnp.log(gh_r / ah_r)
    pxc_at = g_sigx + gi_r
    pyc_at = g_sigy + gj_r
    iou_at = _iou(gx_r, gy_r, gw_r, gh_r, pxc_at, pyc_at, g_pw, g_ph)

    corr = ((g_sigx - txw) ** 2 - (g_sigx - 0.5) ** 2
            + (g_sigy - tyw) ** 2 - (g_sigy - 0.5) ** 2
            + (g_w - tww) ** 2 - g_w * g_w
            + (g_h - thw) ** 2 - g_h * g_h
            + _OBJ * (g_conf - iou_at) ** 2 - mask0_at * g_conf * g_conf)
    corr_sum = jnp.sum(jnp.where(winner_r, corr, 0.0))

    cint = cls_r.astype(i32)
    c_io = lax.broadcasted_iota(i32, (_NC, _NT), 0)
    pick = jnp.sum(jnp.where(c_io == cint, Lg, 0.0), axis=0, keepdims=True)
    m = jnp.max(Lg, axis=0, keepdims=True)
    lse = m + jnp.log(jnp.sum(jnp.exp(Lg - m), axis=0, keepdims=True))
    nll = lse - pick
    cls_sum = jnp.sum(jnp.where(winner_r, nll, 0.0))

    dense_sum = jnp.sum(acc_xy) + jnp.sum(acc_wh) + jnp.sum(acc_conf)
    return (dense_sum + corr_sum) * 0.5 + cls_sum


def _yolo_kernel(o_ref, grid_ref, rrow_ref, rcol_ref, out_ref):
    total = jnp.float32(0.0)
    for i in range(_BIMG):
        total = total + _image_loss(i, o_ref, grid_ref, rrow_ref, rcol_ref)

    @pl.when(pl.program_id(0) == 0)
    def _init():
        out_ref[0] = jnp.zeros((1, 1), jnp.float32)

    out_ref[0] = out_ref[0] + jnp.full((1, 1), total, jnp.float32)


def _grid_consts():
    p = np.arange(_NPIX)
    return np.stack([(p % _NW).astype(np.float32),
                     (p // _NW).astype(np.float32)], axis=0)


def kernel(output, target):
    nB = output.shape[0]
    o = output.reshape(nB, _NA * (5 + _NC), _NPIX)
    gridc = jnp.asarray(_grid_consts())

    rec_row, rec_col = _run_sc_assign(target, nB)

    partial = pl.pallas_call(
        _yolo_kernel,
        grid=(nB // _BIMG,),
        in_specs=[
            pl.BlockSpec((_BIMG, _NA * (5 + _NC), _NPIX),
                         lambda b: (b, 0, 0)),
            pl.BlockSpec((2, _NPIX), lambda b: (0, 0)),
            pl.BlockSpec((_BIMG, _NFIELD, _NTP), lambda b: (b, 0, 0)),
            pl.BlockSpec((_BIMG, _NTP, _NFIELD), lambda b: (b, 0, 0)),
        ],
        out_specs=pl.BlockSpec((1, 1, 1), lambda b: (0, 0, 0)),
        out_shape=jax.ShapeDtypeStruct((1, 1, 1), jnp.float32),
    )(o, gridc, rec_row, rec_col)
    return partial.reshape(())
